# Initial kernel scaffold; baseline (speedup 1.0000x reference)
import jax
import jax.numpy as jnp
from jax.experimental import pallas as pl


def kernel(rel_attn_bias, n):
    raise NotImplementedError("write your pallas kernel here")


# speedup vs baseline: 42.7060x; 42.7060x over previous
import functools

import jax
import jax.numpy as jnp
from jax import lax
from jax.experimental import pallas as pl
from jax.experimental.pallas import tpu as pltpu
from jax.experimental.pallas import tpu_sc as plsc

_HEADS = 16
_N = 2048
_NB = 32
_MAX_EXACT = 8
_P = 8
_W = 4096
_FLAT = _P * _W
_KBLK = 16384


def _table_body(bias_ref, out_ref):
    blk = pl.program_id(0)
    t = lax.broadcasted_iota(jnp.int32, (_NB, _KBLK), 1)
    flat = blk * _KBLK + t
    d = flat // _W
    k = flat - d * _W
    rel = k + (_P - 1) - d - (_N - 1)
    neg = -rel
    sign_add = jnp.where(neg < 0, _NB // 2, 0).astype(jnp.int32)
    na = jnp.abs(neg)
    is_small = na < _MAX_EXACT
    nc = jnp.maximum(na, 1)
    nsq = (nc * nc).astype(jnp.float32)
    e = (lax.bitcast_convert_type(nsq, jnp.int32) >> 23) - 127
    val_large = jnp.minimum(e - 6 + _MAX_EXACT, _NB // 2 - 1)
    bucket = sign_add + jnp.where(is_small, na, val_large)
    b = lax.broadcasted_iota(jnp.int32, (_NB, _KBLK), 0)
    onehot = (bucket == b).astype(jnp.float32)
    out_ref[...] = lax.dot_general(
        bias_ref[...], onehot, (((0,), (0,)), ((), ())),
        precision=lax.Precision.HIGHEST,
        preferred_element_type=jnp.float32)


_table = pl.pallas_call(
    _table_body,
    grid=(_FLAT // _KBLK,),
    in_specs=[pl.BlockSpec((_NB, _HEADS), lambda i: (0, 0))],
    out_specs=pl.BlockSpec((_HEADS, _KBLK), lambda i: (0, i)),
    out_shape=jax.ShapeDtypeStruct((_HEADS, _FLAT), jnp.float32),
)

_NC = 2
_NS = 16
_ROWS_PER_WORKER = _N // 2
_FIRE = 16


def _expand_body(vs_hbm, out_hbm, vs_vmem, sem):
    wid = lax.axis_index("s") * _NC + lax.axis_index("c")
    head = wid // 2
    half = wid % 2
    pltpu.sync_copy(vs_hbm.at[pl.ds(head * _FLAT, _FLAT)], vs_vmem)
    row0 = half * _ROWS_PER_WORKER

    def group(g, carry):
        base = row0 + g * _FIRE
        descs = []
        for u in range(_FIRE):
            i = base + u
            s = (_N - 1) - i
            m = jnp.bitwise_and(s, _P - 1)
            off = pl.multiple_of((_P - 1 - m) * _W + (s - m), _P)
            dst = pl.multiple_of(head * (_N * _N) + i * _N, _N)
            descs.append(pltpu.async_copy(
                vs_vmem.at[pl.ds(off, _N)],
                out_hbm.at[pl.ds(dst, _N)],
                sem))
        for dsc in descs:
            dsc.wait()
        return carry

    lax.fori_loop(0, _ROWS_PER_WORKER // _FIRE, group, 0)


@functools.cache
def _make_expand():
    return functools.partial(
        pl.kernel,
        mesh=plsc.VectorSubcoreMesh(core_axis_name="c", subcore_axis_name="s"),
        out_type=jax.ShapeDtypeStruct((_HEADS * _N * _N,), jnp.float32),
        scratch_types=[
            pltpu.VMEM((_FLAT,), jnp.float32),
            pltpu.SemaphoreType.DMA,
        ],
    )(_expand_body)


def kernel(rel_attn_bias, n):
    del n
    vflat = _table(rel_attn_bias.astype(jnp.float32))
    out = _make_expand()(vflat.reshape(-1))
    return out.reshape(_HEADS, _N, _N)


# speedup vs baseline: 92.5054x; 2.1661x over previous
import functools

import jax
import jax.numpy as jnp
from jax import lax
from jax.experimental import pallas as pl
from jax.experimental.pallas import tpu as pltpu
from jax.experimental.pallas import tpu_sc as plsc

_HEADS = 16
_N = 2048
_NB = 32
_MAX_EXACT = 8
_NP = 16
_W = 4096


def _bucket_values(bias_ref, idx):
    rel = idx - (_N - 1)
    neg = -rel
    sign_add = jnp.where(neg < 0, _NB // 2, 0).astype(jnp.int32)
    na = jnp.abs(neg)
    is_small = na < _MAX_EXACT
    nc = jnp.maximum(na, 1)
    nsq = (nc * nc).astype(jnp.float32)
    e = (lax.bitcast_convert_type(nsq, jnp.int32) >> 23) - 127
    val_large = jnp.minimum(e - 6 + _MAX_EXACT, _NB // 2 - 1)
    bucket = sign_add + jnp.where(is_small, na, val_large)
    b = lax.broadcasted_iota(jnp.int32, bucket.shape, 0)
    onehot = (bucket == b).astype(jnp.float32)
    return lax.dot_general(
        bias_ref[...], onehot, (((0,), (0,)), ((), ())),
        precision=lax.Precision.HIGHEST,
        preferred_element_type=jnp.float32)


def _table_body(bias_ref, out_ref):
    p = pl.program_id(0)
    k = lax.broadcasted_iota(jnp.int32, (_NB, _W), 1)
    for d in range(8):
        res = _bucket_values(bias_ref, k + 127 - 8 * p - d)
        out_ref[0, :, d, :] = res


_table = pl.pallas_call(
    _table_body,
    grid=(_NP,),
    in_specs=[pl.BlockSpec((_NB, _HEADS), lambda i: (0, 0))],
    out_specs=pl.BlockSpec((1, _HEADS, 8, _W), lambda i: (i, 0, 0, 0)),
    out_shape=jax.ShapeDtypeStruct((_NP, _HEADS, 8, _W), jnp.float32),
)

_NC = 2
_NS = 16
_HPW = _HEADS // 2


def _expand_body(a_hbm, out_hbm, buf0, buf1, sem_stage, sem_out):
    wid = lax.axis_index("s") * _NC + lax.axis_index("c")
    p = wid % _NP
    h0 = (wid // _NP) * _HPW
    bufs = (buf0, buf1)

    def stage(h, buf):
        return pltpu.async_copy(a_hbm.at[p, h0 + h], buf, sem_stage)

    pending = stage(0, bufs[0])
    for h in range(_HPW):
        buf = bufs[h % 2]
        pending.wait()
        if h + 1 < _HPW:
            pending = stage(h + 1, bufs[(h + 1) % 2])
        descs = []
        for q in range(_NP):
            rt = _NP * q + p
            row = pl.multiple_of(rt * 8, 8)
            col = pl.multiple_of((_NP - 1 - q) * 128, 128)
            descs.append(pltpu.async_copy(
                buf.at[:, pl.ds(col, _N)],
                out_hbm.at[h0 + h, pl.ds(row, 8), :],
                sem_out))
        for dsc in descs:
            dsc.wait()


@functools.cache
def _make_expand():
    return functools.partial(
        pl.kernel,
        mesh=plsc.VectorSubcoreMesh(core_axis_name="c", subcore_axis_name="s"),
        out_type=jax.ShapeDtypeStruct((_HEADS, _N, _N), jnp.float32),
        scratch_types=[
            pltpu.VMEM((8, _W), jnp.float32),
            pltpu.VMEM((8, _W), jnp.float32),
            pltpu.SemaphoreType.DMA,
            pltpu.SemaphoreType.DMA,
        ],
    )(_expand_body)


def kernel(rel_attn_bias, n):
    del n
    table = _table(rel_attn_bias.astype(jnp.float32))
    return _make_expand()(table)


# speedup vs baseline: 113.8135x; 1.2303x over previous
import functools

import jax
import jax.numpy as jnp
import numpy as np
from jax import lax
from jax.experimental import pallas as pl
from jax.experimental.pallas import tpu as pltpu
from jax.experimental.pallas import tpu_sc as plsc

_HEADS = 16
_N = 2048
_NB = 32
_MAX_EXACT = 8
_NP = 16
_W = 4096
_WIN0 = 1792
_WINW = 512


def _bucket_py(x):
    neg = (_N - 1) - x
    ret = 0 if neg >= 0 else _NB // 2
    na = abs(neg)
    if na < _MAX_EXACT:
        return ret + na
    e = (na * na).bit_length() - 1
    return ret + min(e - 6 + _MAX_EXACT, _NB // 2 - 1)


def _bucket_intervals():
    lo = np.full((_NB, 1), -(2 ** 30), np.int32)
    hi = np.full((_NB, 1), 2 ** 30, np.int32)
    xs = np.arange(-1024, _W + 1024)
    bs = np.array([_bucket_py(int(x)) for x in xs])
    for b in range(_NB):
        sel = xs[bs == b]
        if sel.size:
            if sel.min() > xs.min():
                lo[b, 0] = sel.min()
            if sel.max() < xs.max():
                hi[b, 0] = sel.max()
        else:
            lo[b, 0] = 1
            hi[b, 0] = 0
    return lo, hi


_LO, _HI = _bucket_intervals()
_SEL_1531 = np.zeros((_NB, 2), np.float32)
_SEL_1531[15, 0] = 1.0
_SEL_1531[31, 1] = 1.0


def _table_body(bias_ref, lo_ref, hi_ref, sel_ref, out_ref):
    p = pl.program_id(0)
    lo = lo_ref[...]
    hi = hi_ref[...]
    cb = lax.dot_general(
        bias_ref[...], sel_ref[...], (((0,), (0,)), ((), ())),
        precision=lax.Precision.HIGHEST,
        preferred_element_type=jnp.float32)
    c15 = jnp.broadcast_to(cb[:, 0:1], (_HEADS, _WIN0))
    c31 = jnp.broadcast_to(cb[:, 1:2], (_HEADS, _W - _WIN0 - _WINW))
    kwin = lax.broadcasted_iota(jnp.int32, (_NB, _WINW), 1) + _WIN0
    for d in range(8):
        idx = kwin + (127 - 8 * p - d)
        onehot = ((idx >= lo) & (idx <= hi)).astype(jnp.float32)
        win = lax.dot_general(
            bias_ref[...], onehot, (((0,), (0,)), ((), ())),
            precision=lax.Precision.HIGHEST,
            preferred_element_type=jnp.float32)
        out_ref[0, :, d, 0:_WIN0] = c15
        out_ref[0, :, d, _WIN0:_WIN0 + _WINW] = win
        out_ref[0, :, d, _WIN0 + _WINW:_W] = c31


_table = pl.pallas_call(
    _table_body,
    grid=(_NP,),
    in_specs=[
        pl.BlockSpec((_NB, _HEADS), lambda i: (0, 0)),
        pl.BlockSpec((_NB, 1), lambda i: (0, 0)),
        pl.BlockSpec((_NB, 1), lambda i: (0, 0)),
        pl.BlockSpec((_NB, 2), lambda i: (0, 0)),
    ],
    out_specs=pl.BlockSpec((1, _HEADS, 8, _W), lambda i: (i, 0, 0, 0)),
    out_shape=jax.ShapeDtypeStruct((_NP, _HEADS, 8, _W), jnp.float32),
)

_NC = 2
_NS = 16
_HPW = _HEADS // 2


def _expand_body(a_hbm, out_hbm, buf0, buf1, sem_stage, sem_out):
    wid = lax.axis_index("s") * _NC + lax.axis_index("c")
    p = wid % _NP
    h0 = (wid // _NP) * _HPW
    bufs = (buf0, buf1)

    def stage(h, buf):
        return pltpu.async_copy(a_hbm.at[p, h0 + h], buf, sem_stage)

    pending = stage(0, bufs[0])
    for h in range(_HPW):
        buf = bufs[h % 2]
        pending.wait()
        if h + 1 < _HPW:
            pending = stage(h + 1, bufs[(h + 1) % 2])
        descs = []
        for q in range(_NP):
            rt = _NP * q + p
            row = pl.multiple_of(rt * 8, 8)
            col = pl.multiple_of((_NP - 1 - q) * 128, 128)
            descs.append(pltpu.async_copy(
                buf.at[:, pl.ds(col, _N)],
                out_hbm.at[h0 + h, pl.ds(row, 8), :],
                sem_out))
        for dsc in descs:
            dsc.wait()


@functools.cache
def _make_expand():
    return functools.partial(
        pl.kernel,
        mesh=plsc.VectorSubcoreMesh(core_axis_name="c", subcore_axis_name="s"),
        out_type=jax.ShapeDtypeStruct((_HEADS, _N, _N), jnp.float32),
        scratch_types=[
            pltpu.VMEM((8, _W), jnp.float32),
            pltpu.VMEM((8, _W), jnp.float32),
            pltpu.SemaphoreType.DMA,
            pltpu.SemaphoreType.DMA,
        ],
    )(_expand_body)


def kernel(rel_attn_bias, n):
    del n
    table = _table(rel_attn_bias.astype(jnp.float32), jnp.asarray(_LO),
                   jnp.asarray(_HI), jnp.asarray(_SEL_1531))
    return _make_expand()(table)


# speedup vs baseline: 126.7883x; 1.1140x over previous
import functools

import jax
import jax.numpy as jnp
import numpy as np
from jax import lax
from jax.experimental import pallas as pl
from jax.experimental.pallas import tpu as pltpu
from jax.experimental.pallas import tpu_sc as plsc

_HEADS = 16
_N = 2048
_NB = 32
_MAX_EXACT = 8
_NP = 16
_W = 4096
_WIN0 = 1792
_WINW = 512


def _bucket_py(x):
    neg = (_N - 1) - x
    ret = 0 if neg >= 0 else _NB // 2
    na = abs(neg)
    if na < _MAX_EXACT:
        return ret + na
    e = (na * na).bit_length() - 1
    return ret + min(e - 6 + _MAX_EXACT, _NB // 2 - 1)


def _bucket_intervals():
    lo = np.full((_NB, 1), -(2 ** 30), np.int32)
    hi = np.full((_NB, 1), 2 ** 30, np.int32)
    xs = np.arange(-1024, _W + 1024)
    bs = np.array([_bucket_py(int(x)) for x in xs])
    for b in range(_NB):
        sel = xs[bs == b]
        if sel.size:
            if sel.min() > xs.min():
                lo[b, 0] = sel.min()
            if sel.max() < xs.max():
                hi[b, 0] = sel.max()
        else:
            lo[b, 0] = 1
            hi[b, 0] = 0
    return lo, hi


_LO, _HI = _bucket_intervals()
_SEL_1531 = np.zeros((_NB, 2), np.float32)
_SEL_1531[15, 0] = 1.0
_SEL_1531[31, 1] = 1.0


def _table_body(bias_ref, lo_ref, hi_ref, sel_ref, win_ref, consts_ref):
    p = pl.program_id(0)
    lo = lo_ref[...]
    hi = hi_ref[...]
    cb = lax.dot_general(
        bias_ref[...], sel_ref[...], (((0,), (0,)), ((), ())),
        precision=lax.Precision.HIGHEST,
        preferred_element_type=jnp.float32)
    consts_ref[:, 0:16] = jnp.broadcast_to(cb[:, 0:1], (_HEADS, 16))
    consts_ref[:, 16:32] = jnp.broadcast_to(cb[:, 1:2], (_HEADS, 16))
    kwin = lax.broadcasted_iota(jnp.int32, (_NB, _WINW), 1) + _WIN0
    for d in range(8):
        idx = kwin + (127 - 8 * p - d)
        onehot = ((idx >= lo) & (idx <= hi)).astype(jnp.float32)
        win = lax.dot_general(
            bias_ref[...], onehot, (((0,), (0,)), ((), ())),
            precision=lax.Precision.HIGHEST,
            preferred_element_type=jnp.float32)
        win_ref[0, :, d, :] = win


_table = pl.pallas_call(
    _table_body,
    grid=(_NP,),
    in_specs=[
        pl.BlockSpec((_NB, _HEADS), lambda i: (0, 0)),
        pl.BlockSpec((_NB, 1), lambda i: (0, 0)),
        pl.BlockSpec((_NB, 1), lambda i: (0, 0)),
        pl.BlockSpec((_NB, 2), lambda i: (0, 0)),
    ],
    out_specs=[
        pl.BlockSpec((1, _HEADS, 8, _WINW), lambda i: (i, 0, 0, 0)),
        pl.BlockSpec((_HEADS, 32), lambda i: (0, 0)),
    ],
    out_shape=[
        jax.ShapeDtypeStruct((_NP, _HEADS, 8, _WINW), jnp.float32),
        jax.ShapeDtypeStruct((_HEADS, 32), jnp.float32),
    ],
)

_NC = 2
_NS = 16
_HPW = _HEADS // 2


def _expand_body(win_hbm, c_hbm, out_hbm, buf0, buf1, cbuf,
                 sem_c, sem_stage, sem_out):
    wid = lax.axis_index("s") * _NC + lax.axis_index("c")
    p = wid % _NP
    h0 = (wid // _NP) * _HPW
    bufs = (buf0, buf1)

    def prep(h, buf):
        cd = pltpu.async_copy(
            c_hbm.at[pl.ds(pl.multiple_of((h0 + h) * 32, 8), 32)],
            cbuf, sem_c)
        desc = pltpu.async_copy(
            win_hbm.at[p, h0 + h], buf.at[:, _WIN0:_WIN0 + _WINW], sem_stage)
        cd.wait()
        c15 = cbuf[pl.ds(0, 16)]
        c31 = cbuf[pl.ds(16, 16)]
        for d in range(8):
            row = buf.at[d]

            def left(o, carry, row=row):
                row[pl.ds(pl.multiple_of(o * 16, 16), 16)] = c15
                return carry

            def right(o, carry, row=row):
                row[pl.ds(
                    pl.multiple_of(_WIN0 + _WINW + o * 16, 16), 16)] = c31
                return carry

            lax.fori_loop(0, _WIN0 // 16, left, 0)
            lax.fori_loop(0, (_W - _WIN0 - _WINW) // 16, right, 0)
        return desc

    pending = prep(0, bufs[0])
    for h in range(_HPW):
        buf = bufs[h % 2]
        pending.wait()
        descs = []
        for q in range(_NP):
            rt = _NP * q + p
            row = pl.multiple_of(rt * 8, 8)
            col = pl.multiple_of((_NP - 1 - q) * 128, 128)
            descs.append(pltpu.async_copy(
                buf.at[:, pl.ds(col, _N)],
                out_hbm.at[h0 + h, pl.ds(row, 8), :],
                sem_out))
        if h + 1 < _HPW:
            pending = prep(h + 1, bufs[(h + 1) % 2])
        for dsc in descs:
            dsc.wait()


@functools.cache
def _make_expand():
    return functools.partial(
        pl.kernel,
        mesh=plsc.VectorSubcoreMesh(core_axis_name="c", subcore_axis_name="s"),
        out_type=jax.ShapeDtypeStruct((_HEADS, _N, _N), jnp.float32),
        scratch_types=[
            pltpu.VMEM((8, _W), jnp.float32),
            pltpu.VMEM((8, _W), jnp.float32),
            pltpu.VMEM((32,), jnp.float32),
            pltpu.SemaphoreType.DMA,
            pltpu.SemaphoreType.DMA,
            pltpu.SemaphoreType.DMA,
        ],
    )(_expand_body)


def kernel(rel_attn_bias, n):
    del n
    win, consts = _table(rel_attn_bias.astype(jnp.float32), jnp.asarray(_LO),
                         jnp.asarray(_HI), jnp.asarray(_SEL_1531))
    return _make_expand()(win, consts.reshape(-1))


# speedup vs baseline: 126.9770x; 1.0015x over previous
import functools

import jax
import jax.numpy as jnp
import numpy as np
from jax import lax
from jax.experimental import pallas as pl
from jax.experimental.pallas import tpu as pltpu
from jax.experimental.pallas import tpu_sc as plsc

_HEADS = 16
_N = 2048
_NB = 32
_MAX_EXACT = 8
_NP = 16
_W = 4096
_WIN0 = 1792
_WINW = 512


def _bucket_py(x):
    neg = (_N - 1) - x
    ret = 0 if neg >= 0 else _NB // 2
    na = abs(neg)
    if na < _MAX_EXACT:
        return ret + na
    e = (na * na).bit_length() - 1
    return ret + min(e - 6 + _MAX_EXACT, _NB // 2 - 1)


def _bucket_intervals():
    lo = np.full((_NB, 1), -(2 ** 30), np.int32)
    hi = np.full((_NB, 1), 2 ** 30, np.int32)
    xs = np.arange(-1024, _W + 1024)
    bs = np.array([_bucket_py(int(x)) for x in xs])
    for b in range(_NB):
        sel = xs[bs == b]
        if sel.size:
            if sel.min() > xs.min():
                lo[b, 0] = sel.min()
            if sel.max() < xs.max():
                hi[b, 0] = sel.max()
        else:
            lo[b, 0] = 1
            hi[b, 0] = 0
    return lo, hi


_LO, _HI = _bucket_intervals()
_SEL_1531 = np.zeros((_NB, 2), np.float32)
_SEL_1531[15, 0] = 1.0
_SEL_1531[31, 1] = 1.0


def _table_body(bias_ref, lo_ref, hi_ref, sel_ref, win_ref, consts_ref):
    p = pl.program_id(0)
    lo = lo_ref[...]
    hi = hi_ref[...]
    cb = lax.dot_general(
        bias_ref[...], sel_ref[...], (((0,), (0,)), ((), ())),
        precision=lax.Precision.HIGHEST,
        preferred_element_type=jnp.float32)
    consts_ref[:, 0:16] = jnp.broadcast_to(cb[:, 0:1], (_HEADS, 16))
    consts_ref[:, 16:32] = jnp.broadcast_to(cb[:, 1:2], (_HEADS, 16))
    kwin = lax.broadcasted_iota(jnp.int32, (_NB, _WINW), 1) + _WIN0
    for d in range(8):
        idx = kwin + (127 - 8 * p - d)
        onehot = ((idx >= lo) & (idx <= hi)).astype(jnp.float32)
        win = lax.dot_general(
            bias_ref[...], onehot, (((0,), (0,)), ((), ())),
            precision=lax.Precision.HIGHEST,
            preferred_element_type=jnp.float32)
        win_ref[0, :, d, :] = win


_table = pl.pallas_call(
    _table_body,
    grid=(_NP,),
    in_specs=[
        pl.BlockSpec((_NB, _HEADS), lambda i: (0, 0)),
        pl.BlockSpec((_NB, 1), lambda i: (0, 0)),
        pl.BlockSpec((_NB, 1), lambda i: (0, 0)),
        pl.BlockSpec((_NB, 2), lambda i: (0, 0)),
    ],
    out_specs=[
        pl.BlockSpec((1, _HEADS, 8, _WINW), lambda i: (i, 0, 0, 0)),
        pl.BlockSpec((_HEADS, 32), lambda i: (0, 0)),
    ],
    out_shape=[
        jax.ShapeDtypeStruct((_NP, _HEADS, 8, _WINW), jnp.float32),
        jax.ShapeDtypeStruct((_HEADS, 32), jnp.float32),
    ],
)

_NC = 2
_NS = 16
_HPW = _HEADS // 2


def _expand_body(win_hbm, c_hbm, out_hbm, buf0, buf1, cbuf,
                 sem_c, sem_stage, sem_out):
    wid = lax.axis_index("s") * _NC + lax.axis_index("c")
    p = wid % _NP
    h0 = (wid // _NP) * _HPW
    bufs = (buf0, buf1)

    def prep(h, buf):
        cd = pltpu.async_copy(
            c_hbm.at[pl.ds(pl.multiple_of((h0 + h) * 32, 8), 32)],
            cbuf, sem_c)
        desc = pltpu.async_copy(
            win_hbm.at[p, h0 + h], buf.at[:, _WIN0:_WIN0 + _WINW], sem_stage)
        cd.wait()
        c15 = cbuf[pl.ds(0, 16)]
        c31 = cbuf[pl.ds(16, 16)]
        for d in range(8):
            row = buf.at[d]

            def left(o, carry, row=row):
                row[pl.ds(pl.multiple_of(o * 16, 16), 16)] = c15
                return carry

            def right(o, carry, row=row):
                row[pl.ds(
                    pl.multiple_of(_WIN0 + _WINW + o * 16, 16), 16)] = c31
                return carry

            lax.fori_loop(0, _WIN0 // 16, left, 0)
            lax.fori_loop(0, (_W - _WIN0 - _WINW) // 16, right, 0)
        return desc

    def fire(h, buf):
        descs = []
        for q in range(_NP):
            rt = _NP * q + p
            row = pl.multiple_of(rt * 8, 8)
            col = pl.multiple_of((_NP - 1 - q) * 128, 128)
            descs.append(pltpu.async_copy(
                buf.at[:, pl.ds(col, _N)],
                out_hbm.at[h0 + h, pl.ds(row, 8), :],
                sem_out))
        return descs

    pending = prep(0, bufs[0])
    prev = []
    for h in range(_HPW):
        pending.wait()
        descs = fire(h, bufs[h % 2])
        for dsc in prev:
            dsc.wait()
        if h + 1 < _HPW:
            pending = prep(h + 1, bufs[(h + 1) % 2])
        prev = descs
    for dsc in prev:
        dsc.wait()


@functools.cache
def _make_expand():
    return functools.partial(
        pl.kernel,
        mesh=plsc.VectorSubcoreMesh(core_axis_name="c", subcore_axis_name="s"),
        out_type=jax.ShapeDtypeStruct((_HEADS, _N, _N), jnp.float32),
        scratch_types=[
            pltpu.VMEM((8, _W), jnp.float32),
            pltpu.VMEM((8, _W), jnp.float32),
            pltpu.VMEM((32,), jnp.float32),
            pltpu.SemaphoreType.DMA,
            pltpu.SemaphoreType.DMA,
            pltpu.SemaphoreType.DMA,
        ],
    )(_expand_body)


def kernel(rel_attn_bias, n):
    del n
    win, consts = _table(rel_attn_bias.astype(jnp.float32), jnp.asarray(_LO),
                         jnp.asarray(_HI), jnp.asarray(_SEL_1531))
    return _make_expand()(win, consts.reshape(-1))
